# baseline (speedup 1.0000x reference)
import functools

import jax
import jax.numpy as jnp
from jax import lax
from jax.experimental import pallas as pl
from jax.experimental.pallas import tpu as pltpu
from jax.experimental.pallas import tpu_sc as plsc

D_MODEL = 2048
D_FF = 4096
E = 8
B = 32
LANES = 128
FF_BLK = 512
NF = D_FF // FF_BLK

_SC_INFO = plsc.get_sparse_core_info()
_NC = _SC_INFO.num_cores
_NS = _SC_INFO.num_subcores
_SC_L = _SC_INFO.num_lanes
_NW = _NC * _NS
_NCHUNK = D_MODEL // _SC_L


def _sc_permute(v, idx):
    return lax.gather(
        v, idx[:, None],
        lax.GatherDimensionNumbers(
            offset_dims=(), collapsed_slice_dims=(0,), start_index_map=(0,)),
        (1,), mode=lax.GatherScatterMode.PROMISE_IN_BOUNDS)


def _sc_gate(x_hbm, gwt_hbm, out_hbm, xrow_v, gwt_v, crow_v):
    wid = lax.axis_index("s") * _NC + lax.axis_index("c")
    pltpu.sync_copy(x_hbm.at[wid], xrow_v)
    pltpu.sync_copy(gwt_hbm, gwt_v)

    def body(j, accs):
        off = pl.multiple_of(j * _SC_L, _SC_L)
        xc = xrow_v[pl.ds(off, _SC_L)]
        return tuple(acc + xc * gwt_v[e, pl.ds(off, _SC_L)]
                     for e, acc in enumerate(accs))

    accs = lax.fori_loop(
        0, _NCHUNK, body,
        tuple(jnp.zeros((_SC_L,), jnp.float32) for _ in range(E)))

    iota = lax.broadcasted_iota(jnp.int32, (_SC_L,), 0)

    def lanesum(v):
        for s in (8, 4, 2, 1):
            v = v + _sc_permute(v, iota ^ s)
        return v

    def lanemax(v):
        for s in (8, 4, 2, 1):
            v = jnp.maximum(v, _sc_permute(v, iota ^ s))
        return v

    def lanemin(v):
        for s in (8, 4, 2, 1):
            v = jnp.minimum(v, _sc_permute(v, iota ^ s))
        return v

    neg = jnp.float32(-1e30)
    lv = jnp.full((_SC_L,), neg, jnp.float32)
    for e in range(E):
        lv = jnp.where(iota == e, lanesum(accs[e]), lv)
    m1 = lanemax(lv)
    i1 = lanemin(jnp.where(lv == m1, iota, _SC_L))
    lv2 = jnp.where(iota == i1, neg, lv)
    m2 = lanemax(lv2)
    i2 = lanemin(jnp.where(lv2 == m2, iota, _SC_L))
    z = jnp.exp(m2 - m1)
    p1 = 1.0 / (1.0 + z)
    p2 = 1.0 - p1
    crow_v[...] = (jnp.where(iota == i1, p1, 0.0)
                   + jnp.where(iota == i2, p2, 0.0))
    pltpu.sync_copy(crow_v, out_hbm.at[wid])


def _routing_coeffs(x, gwt):
    k = functools.partial(
        pl.kernel,
        mesh=plsc.VectorSubcoreMesh(core_axis_name="c", subcore_axis_name="s"),
        out_type=jax.ShapeDtypeStruct((B, _SC_L), jnp.float32),
        scratch_types=[
            pltpu.VMEM((D_MODEL,), jnp.float32),
            pltpu.VMEM((E, D_MODEL), jnp.float32),
            pltpu.VMEM((_SC_L,), jnp.float32),
        ],
    )(_sc_gate)
    return k(x, gwt)


def _experts_kernel(x_ref, w1_ref, w3_ref, w2_ref, ys_ref):
    f = pl.program_id(1)
    x = x_ref[...]
    h = jax.nn.silu(jnp.dot(x, w1_ref[0], preferred_element_type=jnp.float32))
    h = h * jnp.dot(x, w3_ref[0], preferred_element_type=jnp.float32)
    y = jnp.dot(h, w2_ref[0], preferred_element_type=jnp.float32)

    @pl.when(f == 0)
    def _init():
        ys_ref[0] = y

    @pl.when(f != 0)
    def _acc():
        ys_ref[0] += y


def _combine_kernel(c_ref, ys_ref, out_ref):
    c = c_ref[...]
    acc = c[:, 0:1] * ys_ref[0]
    for e in range(1, E):
        acc += c[:, e:e + 1] * ys_ref[e]
    out_ref[...] = acc


def kernel(input_i_1SBH, gate_W, w1, w2, w3):
    x = input_i_1SBH.reshape(B, D_MODEL)
    gwt = gate_W.T
    coeff = _routing_coeffs(x, gwt)
    ys = pl.pallas_call(
        _experts_kernel,
        grid=(E, NF),
        in_specs=[
            pl.BlockSpec((B, D_MODEL), lambda e, f: (0, 0)),
            pl.BlockSpec((1, D_MODEL, FF_BLK), lambda e, f: (e, 0, f)),
            pl.BlockSpec((1, D_MODEL, FF_BLK), lambda e, f: (e, 0, f)),
            pl.BlockSpec((1, FF_BLK, D_MODEL), lambda e, f: (e, f, 0)),
        ],
        out_specs=pl.BlockSpec((1, B, D_MODEL), lambda e, f: (e, 0, 0)),
        out_shape=jax.ShapeDtypeStruct((E, B, D_MODEL), jnp.float32),
        compiler_params=pltpu.CompilerParams(
            dimension_semantics=("arbitrary", "arbitrary"),
        ),
    )(x, w1, w3, w2)
    out = pl.pallas_call(
        _combine_kernel,
        in_specs=[
            pl.BlockSpec((B, _SC_L), lambda: (0, 0)),
            pl.BlockSpec((E, B, D_MODEL), lambda: (0, 0, 0)),
        ],
        out_specs=pl.BlockSpec((B, D_MODEL), lambda: (0, 0)),
        out_shape=jax.ShapeDtypeStruct((B, D_MODEL), jnp.float32),
    )(coeff, ys)
    return out.reshape(input_i_1SBH.shape)
